# Initial kernel scaffold; baseline (speedup 1.0000x reference)
import jax
import jax.numpy as jnp
from jax.experimental import pallas as pl


def kernel(x, w_gate, w_noise, gate_threshold, experts_mask, noise, W1, b1, W2, b2):
    raise NotImplementedError("write your pallas kernel here")


# speedup vs baseline: 2.0646x; 2.0646x over previous
import functools

import jax
import jax.numpy as jnp
from jax.experimental import pallas as pl


def _fused_body(x_ref, noise_ref, wgn_ref, thr_ref, msk_ref,
                w1c_ref, b1_ref, w2s_ref, b2_ref, out_ref, *, E, D):
    xf = x_ref[...]
    g8 = jnp.dot(xf, wgn_ref[...], preferred_element_type=jnp.float32)
    clean = g8[:, :E]
    rawn = g8[:, E:2 * E]
    noise_std = jax.nn.softplus(rawn) + 1e-2
    noisy = clean + noise_ref[...] * noise_std
    scores = noisy - thr_ref[...]
    sel = 0.5 * (jnp.sign(scores) + 1.0) * msk_ref[...]
    masked = jnp.where(sel > 0.0, clean, jnp.full_like(clean, -1e9))
    m = jnp.max(masked, axis=-1, keepdims=True)
    ex = jnp.exp(masked - m)
    sm = ex / jnp.sum(ex, axis=-1, keepdims=True)
    gates = sm * sel
    denom = jnp.clip(jnp.sum(gates, axis=-1, keepdims=True), 1e-9, None)
    gates = gates / denom
    xb = xf.astype(jnp.bfloat16)
    h = jnp.dot(xb, w1c_ref[...], preferred_element_type=jnp.float32)
    parts = []
    bias = None
    for e in range(E):
        he = h[:, e * D:(e + 1) * D] + b1_ref[e][None, :]
        ge = gates[:, e:e + 1]
        parts.append((jnp.maximum(he, 0.0) * ge).astype(jnp.bfloat16))
        be = ge * b2_ref[e][None, :]
        bias = be if bias is None else bias + be
    hg = jnp.concatenate(parts, axis=1)
    out = jnp.dot(hg, w2s_ref[...], preferred_element_type=jnp.float32)
    out_ref[...] = out + bias


def kernel(x, w_gate, w_noise, gate_threshold, experts_mask, noise, W1, b1, W2, b2):
    N, D = x.shape
    E = w_gate.shape[1]
    BT = 256
    gw = jnp.concatenate([w_gate, w_noise], axis=1)
    wgn = jnp.pad(gw, ((0, 0), (0, 128 - 2 * E)))
    w1c = jnp.transpose(W1, (1, 0, 2)).reshape(D, E * D).astype(jnp.bfloat16)
    w2s = W2.reshape(E * D, D).astype(jnp.bfloat16)
    thr = gate_threshold.reshape(1, E)
    msk = experts_mask.reshape(1, E)

    grid = (N // BT,)
    body = functools.partial(_fused_body, E=E, D=D)
    return pl.pallas_call(
        body,
        grid=grid,
        in_specs=[
            pl.BlockSpec((BT, D), lambda i: (i, 0)),
            pl.BlockSpec((BT, E), lambda i: (i, 0)),
            pl.BlockSpec((D, 128), lambda i: (0, 0)),
            pl.BlockSpec((1, E), lambda i: (0, 0)),
            pl.BlockSpec((1, E), lambda i: (0, 0)),
            pl.BlockSpec((D, E * D), lambda i: (0, 0)),
            pl.BlockSpec((E, D), lambda i: (0, 0)),
            pl.BlockSpec((E * D, D), lambda i: (0, 0)),
            pl.BlockSpec((E, D), lambda i: (0, 0)),
        ],
        out_specs=pl.BlockSpec((BT, D), lambda i: (i, 0)),
        out_shape=jax.ShapeDtypeStruct((N, D), jnp.float32),
    )(x, noise, wgn, thr, msk, w1c, b1, w2s, b2)


# speedup vs baseline: 2.0785x; 1.0067x over previous
import functools

import jax
import jax.numpy as jnp
from jax.experimental import pallas as pl


def _fused_body(x_ref, noise_ref, wgn_ref, thr_ref, msk_ref,
                w1c_ref, b1_ref, w2s_ref, out_ref, *, E, D):
    xf = x_ref[...]
    g8 = jnp.dot(xf, wgn_ref[...], preferred_element_type=jnp.float32)
    clean = g8[:, :E]
    rawn = g8[:, E:2 * E]
    noise_std = jax.nn.softplus(rawn) + 1e-2
    noisy = clean + noise_ref[...] * noise_std
    scores = noisy - thr_ref[...]
    sel = 0.5 * (jnp.sign(scores) + 1.0) * msk_ref[...]
    masked = jnp.where(sel > 0.0, clean, jnp.full_like(clean, -1e9))
    m = jnp.max(masked, axis=-1, keepdims=True)
    ex = jnp.exp(masked - m)
    sm = ex / jnp.sum(ex, axis=-1, keepdims=True)
    gates = sm * sel
    denom = jnp.clip(jnp.sum(gates, axis=-1, keepdims=True), 1e-9, None)
    gates = gates / denom
    xb = xf.astype(jnp.bfloat16)
    h = jnp.dot(xb, w1c_ref[...], preferred_element_type=jnp.float32)
    parts = []
    for e in range(E):
        he = h[:, e * D:(e + 1) * D] + b1_ref[e][None, :]
        ge = gates[:, e:e + 1]
        parts.append((jnp.maximum(he, 0.0) * ge).astype(jnp.bfloat16))
    gpad = jnp.pad(gates, ((0, 0), (0, 128 - E))).astype(jnp.bfloat16)
    parts.append(gpad)
    hg = jnp.concatenate(parts, axis=1)
    out_ref[...] = jnp.dot(hg, w2s_ref[...], preferred_element_type=jnp.float32)


def kernel(x, w_gate, w_noise, gate_threshold, experts_mask, noise, W1, b1, W2, b2):
    N, D = x.shape
    E = w_gate.shape[1]
    BT = 512
    gw = jnp.concatenate([w_gate, w_noise], axis=1)
    wgn = jnp.pad(gw, ((0, 0), (0, 128 - 2 * E)))
    w1c = jnp.transpose(W1, (1, 0, 2)).reshape(D, E * D).astype(jnp.bfloat16)
    w2s = jnp.concatenate(
        [W2.reshape(E * D, D), jnp.pad(b2, ((0, 128 - E), (0, 0)))],
        axis=0).astype(jnp.bfloat16)
    thr = gate_threshold.reshape(1, E)
    msk = experts_mask.reshape(1, E)

    grid = (N // BT,)
    body = functools.partial(_fused_body, E=E, D=D)
    return pl.pallas_call(
        body,
        grid=grid,
        in_specs=[
            pl.BlockSpec((BT, D), lambda i: (i, 0)),
            pl.BlockSpec((BT, E), lambda i: (i, 0)),
            pl.BlockSpec((D, 128), lambda i: (0, 0)),
            pl.BlockSpec((1, E), lambda i: (0, 0)),
            pl.BlockSpec((1, E), lambda i: (0, 0)),
            pl.BlockSpec((D, E * D), lambda i: (0, 0)),
            pl.BlockSpec((E, D), lambda i: (0, 0)),
            pl.BlockSpec((E * D + 128, D), lambda i: (0, 0)),
        ],
        out_specs=pl.BlockSpec((BT, D), lambda i: (i, 0)),
        out_shape=jax.ShapeDtypeStruct((N, D), jnp.float32),
    )(x, noise, wgn, thr, msk, w1c, b1, w2s)


# speedup vs baseline: 2.2245x; 1.0703x over previous
import functools

import jax
import jax.numpy as jnp
from jax.experimental import pallas as pl


def _fused_body(x_ref, noise_ref, wgn_ref, thr_ref, msk_ref,
                w1_ref, b1_ref, w2_ref, b2p_ref, out_ref, *, E, D):
    xf = x_ref[...]
    g8 = jnp.dot(xf, wgn_ref[...], preferred_element_type=jnp.float32)
    clean = g8[:, :E]
    rawn = g8[:, E:2 * E]
    noise_std = jax.nn.softplus(rawn) + 1e-2
    noisy = clean + noise_ref[...] * noise_std
    scores = noisy - thr_ref[...]
    sel = 0.5 * (jnp.sign(scores) + 1.0) * msk_ref[...]
    masked = jnp.where(sel > 0.0, clean, jnp.full_like(clean, -1e9))
    m = jnp.max(masked, axis=-1, keepdims=True)
    ex = jnp.exp(masked - m)
    sm = ex / jnp.sum(ex, axis=-1, keepdims=True)
    gates = sm * sel
    denom = jnp.clip(jnp.sum(gates, axis=-1, keepdims=True), 1e-9, None)
    gates = gates / denom
    xb = xf.astype(jnp.bfloat16)
    gpad = jnp.pad(gates, ((0, 0), (0, 128 - E)))
    acc = jnp.dot(gpad, b2p_ref[...], preferred_element_type=jnp.float32)
    for e in range(E):
        he = jnp.dot(xb, w1_ref[e], preferred_element_type=jnp.float32)
        he = he + b1_ref[e][None, :]
        hg = (jnp.maximum(he, 0.0) * gates[:, e:e + 1]).astype(jnp.bfloat16)
        acc = acc + jnp.dot(hg, w2_ref[e], preferred_element_type=jnp.float32)
    out_ref[...] = acc


def kernel(x, w_gate, w_noise, gate_threshold, experts_mask, noise, W1, b1, W2, b2):
    N, D = x.shape
    E = w_gate.shape[1]
    BT = 512
    gw = jnp.concatenate([w_gate, w_noise], axis=1)
    wgn = jnp.pad(gw, ((0, 0), (0, 128 - 2 * E)))
    w1b = W1.astype(jnp.bfloat16)
    w2b = W2.astype(jnp.bfloat16)
    b2p = jnp.pad(b2, ((0, 128 - E), (0, 0)))
    thr = gate_threshold.reshape(1, E)
    msk = experts_mask.reshape(1, E)

    grid = (N // BT,)
    body = functools.partial(_fused_body, E=E, D=D)
    return pl.pallas_call(
        body,
        grid=grid,
        in_specs=[
            pl.BlockSpec((BT, D), lambda i: (i, 0)),
            pl.BlockSpec((BT, E), lambda i: (i, 0)),
            pl.BlockSpec((D, 128), lambda i: (0, 0)),
            pl.BlockSpec((1, E), lambda i: (0, 0)),
            pl.BlockSpec((1, E), lambda i: (0, 0)),
            pl.BlockSpec((E, D, D), lambda i: (0, 0, 0)),
            pl.BlockSpec((E, D), lambda i: (0, 0)),
            pl.BlockSpec((E, D, D), lambda i: (0, 0, 0)),
            pl.BlockSpec((128, D), lambda i: (0, 0)),
        ],
        out_specs=pl.BlockSpec((BT, D), lambda i: (i, 0)),
        out_shape=jax.ShapeDtypeStruct((N, D), jnp.float32),
    )(x, noise, wgn, thr, msk, w1b, b1, w2b, b2p)
